# baseline (speedup 1.0000x reference)
import functools

import jax
import jax.numpy as jnp
from jax import lax
from jax.experimental import pallas as pl
from jax.experimental.pallas import tpu as pltpu
from jax.experimental.pallas import tpu_sc as plsc

_NC = 2
_NS = 16
_CE = 512 * 1024
_NR = 3


def kernel(inputs, table):
    B, S, D = inputs.shape
    N = B * S * D
    half = N // _NC
    chunks = half // _CE

    xf = inputs.reshape(N)
    tf = table.reshape(S * D)

    mesh = plsc.VectorSubcoreMesh(core_axis_name="c", subcore_axis_name="s")

    scratch = (
        [pltpu.VMEM_SHARED((_CE,), jnp.float32) for _ in range(_NR)]
        + [pltpu.SemaphoreType.DMA] * (2 * _NR)
    )

    @functools.partial(
        pl.kernel,
        out_type=jax.ShapeDtypeStruct((N,), jnp.float32),
        mesh=mesh,
        scratch_types=scratch,
    )
    def sc_pump(x_hbm, t_hbm, o_hbm, *bufs):
        sb = bufs[:_NR]
        in_sem = bufs[_NR:2 * _NR]
        out_sem = bufs[2 * _NR:]

        cid = lax.axis_index("c")
        sid = lax.axis_index("s")
        base = cid * half

        def start_in(k):
            r = k % _NR
            return pltpu.async_copy(
                x_hbm.at[pl.ds(base + k * _CE, _CE)], sb[r], in_sem[r])

        def start_out(k):
            r = k % _NR
            return pltpu.async_copy(
                sb[r], o_hbm.at[pl.ds(base + k * _CE, _CE)], out_sem[r])

        @pl.when(sid == 0)
        def _pump():
            in_d = {}
            out_d = {}
            for k in range(min(2, chunks)):
                in_d[k] = start_in(k)
            for k in range(chunks):
                v = k + 2
                if v < chunks:
                    if v - _NR >= 0:
                        out_d[v - _NR].wait()
                    in_d[v] = start_in(v)
                in_d[k].wait()
                out_d[k] = start_out(k)
            for k in range(max(0, chunks - _NR), chunks):
                out_d[k].wait()

    out = sc_pump(xf, tf)
    return out.reshape(B, S, D)


# speedup vs baseline: 1.0364x; 1.0364x over previous
import functools

import jax
import jax.numpy as jnp
from jax import lax
from jax.experimental import pallas as pl
from jax.experimental.pallas import tpu as pltpu
from jax.experimental.pallas import tpu_sc as plsc

_NC = 2
_NS = 16
_CE = 512 * 1024
_NR = 3


def kernel(inputs, table):
    B, S, D = inputs.shape
    N = B * S * D
    half = N // _NC
    chunks = half // _CE

    xf = inputs.reshape(N)
    tf = table.reshape(S * D)

    mesh = plsc.VectorSubcoreMesh(core_axis_name="c", subcore_axis_name="s")

    scratch = (
        [pltpu.VMEM_SHARED((_CE,), jnp.float32) for _ in range(_NR)]
        + [pltpu.SemaphoreType.DMA] * (2 * _NR)
    )

    @functools.partial(
        pl.kernel,
        out_type=jax.ShapeDtypeStruct((N,), jnp.float32),
        mesh=mesh,
        scratch_types=scratch,
    )
    def sc_pump(x_hbm, t_hbm, o_hbm, *bufs):
        sb = bufs[:_NR]
        in_sem = bufs[_NR:2 * _NR]
        out_sem = bufs[2 * _NR:]

        cid = lax.axis_index("c")
        sid = lax.axis_index("s")
        sl = _CE // _NS
        base = cid * half + sid * sl

        def start_in(k):
            r = k % _NR
            return pltpu.async_copy(
                x_hbm.at[pl.ds(base + k * _CE, sl)],
                sb[r].at[pl.ds(sid * sl, sl)], in_sem[r])

        def start_out(k):
            r = k % _NR
            return pltpu.async_copy(
                sb[r].at[pl.ds(sid * sl, sl)],
                o_hbm.at[pl.ds(base + k * _CE, sl)], out_sem[r])

        in_d = {}
        out_d = {}
        for k in range(min(2, chunks)):
            in_d[k] = start_in(k)
        for k in range(chunks):
            v = k + 2
            if v < chunks:
                if v - _NR >= 0:
                    out_d[v - _NR].wait()
                in_d[v] = start_in(v)
            in_d[k].wait()
            out_d[k] = start_out(k)
        for k in range(max(0, chunks - _NR), chunks):
            out_d[k].wait()

    out = sc_pump(xf, tf)
    return out.reshape(B, S, D)


# speedup vs baseline: 1.6613x; 1.6030x over previous
import functools

import jax
import jax.numpy as jnp
from jax import lax
from jax.experimental import pallas as pl
from jax.experimental.pallas import tpu as pltpu
from jax.experimental.pallas import tpu_sc as plsc

_NC = 2
_NS = 16
_NW = _NC * _NS
_TS = 16
_NXB = 2
_NTB = 2
_S_SC = 1024
_BS = 1024


def _sc_part(inputs, table):
    B, S, D = inputs.shape
    rows_w = _S_SC // _NW
    tiles_w = rows_w // _TS
    tile_e = _TS * D

    x4 = inputs.reshape(B, S * D)
    tf = table.reshape(S * D)

    mesh = plsc.VectorSubcoreMesh(core_axis_name="c", subcore_axis_name="s")

    scratch = (
        [pltpu.VMEM((B, tile_e), jnp.float32) for _ in range(_NXB)]
        + [pltpu.VMEM((tile_e,), jnp.float32) for _ in range(_NTB)]
        + [pltpu.SemaphoreType.DMA] * (2 * _NXB + _NTB)
    )

    @functools.partial(
        pl.kernel,
        out_type=jax.ShapeDtypeStruct((B, _S_SC * D), jnp.float32),
        mesh=mesh,
        scratch_types=scratch,
    )
    def sc_add(x_hbm, t_hbm, o_hbm, *bufs):
        xb = bufs[:_NXB]
        tb = bufs[_NXB:_NXB + _NTB]
        xin_sem = bufs[_NXB + _NTB:2 * _NXB + _NTB]
        xout_sem = bufs[2 * _NXB + _NTB:3 * _NXB + _NTB]
        tin_sem = bufs[3 * _NXB + _NTB:]

        wid = lax.axis_index("s") * _NC + lax.axis_index("c")
        base = wid * rows_w * D

        def start_in(t):
            p = t % _NXB
            return pltpu.async_copy(
                x_hbm.at[:, pl.ds(base + t * tile_e, tile_e)], xb[p],
                xin_sem[p])

        def start_tab(t):
            q = t % _NTB
            return pltpu.async_copy(
                t_hbm.at[pl.ds(base + t * tile_e, tile_e)], tb[q], tin_sem[q])

        in_d = {}
        out_d = {}
        tab_d = {}
        for t in range(min(_NTB, tiles_w)):
            tab_d[t] = start_tab(t)
        in_d[0] = start_in(0)

        for t in range(tiles_w):
            p = t % _NXB

            v = t + 1
            if v < tiles_w:
                if v - _NXB >= 0:
                    out_d[v - _NXB].wait()
                in_d[v] = start_in(v)

            tab_d[t].wait()
            in_d[t].wait()

            tbq = tb[t % _NTB]
            xbp = xb[p]

            @plsc.parallel_loop(0, tile_e, step=16, unroll=8)
            def _add(i):
                for b in range(B):
                    xbp[b, pl.ds(i, 16)] = xbp[b, pl.ds(i, 16)] + tbq[pl.ds(i, 16)]

            out_d[t] = pltpu.async_copy(
                xbp, o_hbm.at[:, pl.ds(base + t * tile_e, tile_e)],
                xout_sem[p])

            if t + _NTB < tiles_w:
                tab_d[t + _NTB] = start_tab(t + _NTB)

        for t in range(max(0, tiles_w - _NXB), tiles_w):
            out_d[t].wait()

    return sc_add(x4, tf)


def _tc_body(x_ref, t_ref, o_ref):
    o_ref[...] = x_ref[...] + t_ref[...]


def _tc_part(inputs, table):
    B, S, D = inputs.shape
    nb0 = _S_SC // _BS
    grid = ((S - _S_SC) // _BS, B)
    return pl.pallas_call(
        _tc_body,
        grid=grid,
        in_specs=[
            pl.BlockSpec((1, _BS, D), lambda i, j: (j, nb0 + i, 0)),
            pl.BlockSpec((_BS, D), lambda i, j: (nb0 + i, 0)),
        ],
        out_specs=pl.BlockSpec((1, _BS, D), lambda i, j: (j, nb0 + i, 0)),
        out_shape=jax.ShapeDtypeStruct((B, S, D), inputs.dtype),
    )(inputs, table)


def kernel(inputs, table):
    B, S, D = inputs.shape
    sc_out = _sc_part(inputs, table).reshape(B, _S_SC, D)
    tc_out = _tc_part(inputs, table)
    return lax.dynamic_update_slice(tc_out, sc_out, (0, 0, 0))
